# baseline (speedup 1.0000x reference)
import functools

import jax
import jax.numpy as jnp
from jax import lax
from jax.experimental import pallas as pl
from jax.experimental.pallas import tpu as pltpu
from jax.experimental.pallas import tpu_sc as plsc

_D = 64
_G = 128
_U = 5
_NBUF = 2

_info = plsc.get_sparse_core_info()
_NC = _info.num_cores
_NS = _info.num_subcores
_NW = _NC * _NS


_TBLK = 8192


def _ttab_body(in_ref, out_ref):
    out_ref[...] = in_ref[...].T


def _tc_transpose_table(t_t):
    v = t_t.shape[1]
    return pl.pallas_call(
        _ttab_body,
        grid=(pl.cdiv(v, _TBLK),),
        in_specs=[pl.BlockSpec((_D, _TBLK), lambda i: (0, i))],
        out_specs=pl.BlockSpec((_TBLK, _D), lambda i: (i, 0)),
        out_shape=jax.ShapeDtypeStruct((v, _D), jnp.float32),
    )(t_t)


def _tout_body(in_ref, out_ref):
    out_ref[0] = in_ref[0].T


def _tc_untranspose_out(flat, s, b):
    x3 = flat.reshape(s, b, _D)
    return pl.pallas_call(
        _tout_body,
        grid=(s,),
        in_specs=[pl.BlockSpec((1, b, _D), lambda i: (i, 0, 0))],
        out_specs=pl.BlockSpec((1, _D, b), lambda i: (i, 0, 0)),
        out_shape=jax.ShapeDtypeStruct((s, _D, b), jnp.float32),
    )(x3)


def _make_gather(n_rows: int):
    rows_per_w = n_rows // _NW
    n_chunks = rows_per_w // _U
    assert n_chunks % _NBUF == 0
    mesh = plsc.VectorSubcoreMesh(core_axis_name="c", subcore_axis_name="s")

    @functools.partial(
        pl.kernel,
        mesh=mesh,
        out_type=jax.ShapeDtypeStruct((n_rows * _G, _D), jnp.float32),
        scratch_types=[
            pltpu.VMEM((rows_per_w, _G), jnp.int32),
            pltpu.VMEM((_NBUF, _U * _G, _D), jnp.float32),
            pltpu.SemaphoreType.DMA((_NBUF,)),
            pltpu.SemaphoreType.DMA((_NBUF,)),
        ],
        compiler_params=pltpu.CompilerParams(use_tc_tiling_on_sc=False),
    )
    def k(table_hbm, idx_hbm, out_hbm, idx_v, rows_v, gsem, osem):
        wid = lax.axis_index("s") * _NC + lax.axis_index("c")
        w_row0 = wid * rows_per_w

        def fire_gathers(c, buf):
            for j in range(_U):
                pltpu.async_copy(
                    table_hbm.at[idx_v.at[c * _U + j]],
                    rows_v.at[buf, pl.ds(j * _G, _G)],
                    gsem.at[buf],
                )

        def drain_gathers(buf):
            for j in range(_U):
                pltpu.make_async_copy(
                    table_hbm.at[idx_v.at[0]],
                    rows_v.at[buf, pl.ds(j * _G, _G)],
                    gsem.at[buf],
                ).wait()

        def fire_write(c, buf):
            pltpu.async_copy(
                rows_v.at[buf],
                out_hbm.at[pl.ds((w_row0 + c * _U) * _G, _U * _G)],
                osem.at[buf],
            )

        def drain_write(buf):
            pltpu.make_async_copy(
                rows_v.at[buf],
                out_hbm.at[pl.ds(w_row0 * _G, _U * _G)],
                osem.at[buf],
            ).wait()

        pltpu.sync_copy(idx_hbm.at[pl.ds(w_row0, rows_per_w)], idx_v)
        fire_gathers(0, 0)

        @pl.loop(0, n_chunks, step=_NBUF)
        def trip(g):
            @pl.when(g > 0)
            def _():
                drain_write(1)
            fire_gathers(g + 1, 1)
            drain_gathers(0)
            fire_write(g, 0)

            @pl.when(g < n_chunks - _NBUF)
            def _():
                drain_write(0)
                fire_gathers(g + 2, 0)
            drain_gathers(1)
            fire_write(g + 1, 1)

        drain_write(0)
        drain_write(1)

    return k


def kernel(x, table):
    b, s = x.shape
    x_t = jnp.transpose(x)
    t_t = jnp.transpose(table)
    table_rm = _tc_transpose_table(t_t)
    x2d = x_t.reshape(-1, _G)
    flat = _make_gather(x2d.shape[0])(table_rm, x2d)
    out_t = _tc_untranspose_out(flat, s, b)
    return jnp.transpose(out_t, (2, 0, 1))


# speedup vs baseline: 1.5653x; 1.5653x over previous
import functools

import jax
import jax.numpy as jnp
from jax import lax
from jax.experimental import pallas as pl
from jax.experimental.pallas import tpu as pltpu
from jax.experimental.pallas import tpu_sc as plsc

_D = 64
_G = 128
_U = 5
_NBUF = 2
_P = 8192
_PB = _P.bit_length() - 1

_info = plsc.get_sparse_core_info()
_NC = _info.num_cores
_NS = _info.num_subcores
_NW = _NC * _NS


def _ttab_body(in_ref, out_ref):
    out_ref[:, :_D] = in_ref[:, :_P].T
    out_ref[:, _D:] = in_ref[:, _P:].T


def _tc_transpose_table(t_t):
    v = t_t.shape[1]
    n_blk = pl.cdiv(v, 2 * _P)
    return pl.pallas_call(
        _ttab_body,
        grid=(n_blk,),
        in_specs=[pl.BlockSpec((_D, 2 * _P), lambda i: (0, i))],
        out_specs=pl.BlockSpec((_P, 2 * _D), lambda i: (i, 0)),
        out_shape=jax.ShapeDtypeStruct((n_blk * _P, 2 * _D), jnp.float32),
        compiler_params=pltpu.CompilerParams(
            dimension_semantics=("parallel",),
        ),
    )(t_t)


def _tout_body(in_ref, out_ref):
    h = in_ref.shape[0]
    out_ref[0, :, :h] = in_ref[:, :_D].T
    out_ref[0, :, h:] = in_ref[:, _D:].T


def _tc_untranspose_out(flat128, s, b):
    h = b // 2
    return pl.pallas_call(
        _tout_body,
        grid=(s,),
        in_specs=[pl.BlockSpec((h, 2 * _D), lambda i: (i, 0))],
        out_specs=pl.BlockSpec((1, _D, b), lambda i: (i, 0, 0)),
        out_shape=jax.ShapeDtypeStruct((s, _D, b), jnp.float32),
        compiler_params=pltpu.CompilerParams(
            dimension_semantics=("parallel",),
        ),
    )(flat128)


def _make_gather(n_rows: int):
    rows_per_w = n_rows // _NW
    n_chunks = rows_per_w // _U
    assert n_chunks % _NBUF == 0
    mesh = plsc.VectorSubcoreMesh(core_axis_name="c", subcore_axis_name="s")

    @functools.partial(
        pl.kernel,
        mesh=mesh,
        out_type=jax.ShapeDtypeStruct((n_rows * _G, _D), jnp.float32),
        scratch_types=[
            pltpu.VMEM((rows_per_w, _G), jnp.int32),
            pltpu.VMEM((_NBUF, _U * _G, _D), jnp.float32),
            pltpu.SemaphoreType.DMA((_NBUF,)),
            pltpu.SemaphoreType.DMA((_NBUF,)),
        ],
        compiler_params=pltpu.CompilerParams(use_tc_tiling_on_sc=False),
    )
    def k(table_hbm, idx_hbm, out_hbm, idx_v, rows_v, gsem, osem):
        wid = lax.axis_index("s") * _NC + lax.axis_index("c")
        w_row0 = wid * rows_per_w

        def fire_gathers(c, buf):
            for j in range(_U):
                pltpu.async_copy(
                    table_hbm.at[idx_v.at[c * _U + j]],
                    rows_v.at[buf, pl.ds(j * _G, _G)],
                    gsem.at[buf],
                )

        def drain_gathers(buf):
            for j in range(_U):
                pltpu.make_async_copy(
                    table_hbm.at[idx_v.at[0]],
                    rows_v.at[buf, pl.ds(j * _G, _G)],
                    gsem.at[buf],
                ).wait()

        def fire_write(c, buf):
            pltpu.async_copy(
                rows_v.at[buf],
                out_hbm.at[pl.ds((w_row0 + c * _U) * _G, _U * _G)],
                osem.at[buf],
            )

        def drain_write(buf):
            pltpu.make_async_copy(
                rows_v.at[buf],
                out_hbm.at[pl.ds(w_row0 * _G, _U * _G)],
                osem.at[buf],
            ).wait()

        pltpu.sync_copy(idx_hbm.at[pl.ds(w_row0, rows_per_w)], idx_v)
        fire_gathers(0, 0)

        @pl.loop(0, n_chunks, step=_NBUF)
        def trip(g):
            @pl.when(g > 0)
            def _():
                drain_write(1)
            fire_gathers(g + 1, 1)
            drain_gathers(0)
            fire_write(g, 0)

            @pl.when(g < n_chunks - _NBUF)
            def _():
                drain_write(0)
                fire_gathers(g + 2, 0)
            drain_gathers(1)
            fire_write(g + 1, 1)

        drain_write(0)
        drain_write(1)

    return k


def kernel(x, table):
    b, s = x.shape
    h = b // 2
    x_t = jnp.transpose(x)
    t_t = jnp.transpose(table)

    table128 = _tc_transpose_table(t_t)
    table_rm = table128.reshape(-1, _D)

    xp = jnp.transpose(x_t.reshape(s, 2, h), (0, 2, 1)).reshape(s, b)
    blk = xp >> (_PB + 1)
    u = xp & (2 * _P - 1)
    xg = (blk << (_PB + 1)) + ((u & (_P - 1)) << 1) + (u >> _PB)

    x2d = xg.reshape(-1, _G)
    flat = _make_gather(x2d.shape[0])(table_rm, x2d)
    flat128 = flat.reshape(-1, 2 * _D)
    out_t = _tc_untranspose_out(flat128, s, b)
    return jnp.transpose(out_t, (2, 0, 1))


# speedup vs baseline: 2.1361x; 1.3647x over previous
import functools

import jax
import jax.numpy as jnp
from jax import lax
from jax.experimental import pallas as pl
from jax.experimental.pallas import tpu as pltpu
from jax.experimental.pallas import tpu_sc as plsc

_D = 64
_G = 128
_U = 4
_NBUF = 2
_P = 8192
_PB = _P.bit_length() - 1

_info = plsc.get_sparse_core_info()
_NC = _info.num_cores
_NS = _info.num_subcores
_NW = _NC * _NS


def _ttab_body(in_ref, out_ref):
    out_ref[:, :_D] = in_ref[:, :_P].T
    out_ref[:, _D:] = in_ref[:, _P:].T


def _tc_transpose_table(t_t):
    v = t_t.shape[1]
    n_blk = pl.cdiv(v, 2 * _P)
    return pl.pallas_call(
        _ttab_body,
        grid=(n_blk,),
        in_specs=[pl.BlockSpec((_D, 2 * _P), lambda i: (0, i))],
        out_specs=pl.BlockSpec((_P, 2 * _D), lambda i: (i, 0)),
        out_shape=jax.ShapeDtypeStruct((n_blk * _P, 2 * _D), jnp.float32),
        compiler_params=pltpu.CompilerParams(
            dimension_semantics=("parallel",),
        ),
    )(t_t)


def _tout_body(in_ref, out_ref):
    h = in_ref.shape[0]
    out_ref[0, :, :h] = in_ref[:, :_D].T
    out_ref[0, :, h:] = in_ref[:, _D:].T


def _tc_untranspose_out(flat128, s, b):
    h = b // 2
    return pl.pallas_call(
        _tout_body,
        grid=(s,),
        in_specs=[pl.BlockSpec((h, 2 * _D), lambda i: (i, 0))],
        out_specs=pl.BlockSpec((1, _D, b), lambda i: (i, 0, 0)),
        out_shape=jax.ShapeDtypeStruct((s, _D, b), jnp.float32),
        compiler_params=pltpu.CompilerParams(
            dimension_semantics=("parallel",),
        ),
    )(flat128)


def _make_gather(n_rows: int):
    rows_per_w = n_rows // _NW
    n_chunks = rows_per_w // _U
    assert n_chunks % _NBUF == 0
    mesh = plsc.VectorSubcoreMesh(core_axis_name="c", subcore_axis_name="s")

    chunk = _U * _G

    @functools.partial(
        pl.kernel,
        mesh=mesh,
        out_type=jax.ShapeDtypeStruct((n_rows * _G // 2, 2 * _D), jnp.float32),
        scratch_types=[
            pltpu.VMEM((rows_per_w, _G), jnp.int32),
            pltpu.VMEM((_NBUF, _U * _G, _D), jnp.float32),
            pltpu.SemaphoreType.DMA((_NBUF,)),
            pltpu.SemaphoreType.DMA((_NBUF,)),
        ],
        compiler_params=pltpu.CompilerParams(use_tc_tiling_on_sc=False),
    )
    def k(table_hbm, idx_hbm, out_hbm, idx_v, rows_v, gsem, osem):
        wid = lax.axis_index("s") * _NC + lax.axis_index("c")
        w_row0 = wid * rows_per_w

        def fire_gathers(c, buf):
            for j in range(_U):
                pltpu.async_copy(
                    table_hbm.at[idx_v.at[c * _U + j]],
                    rows_v.at[buf, pl.ds(j * _G, _G)],
                    gsem.at[buf],
                )

        def drain_gathers(buf):
            for j in range(_U):
                pltpu.make_async_copy(
                    table_hbm.at[idx_v.at[0]],
                    rows_v.at[buf, pl.ds(j * _G, _G)],
                    gsem.at[buf],
                ).wait()

        def fire_write(c, buf):
            t0 = (w_row0 + c * _U) * _G
            r0 = ((t0 >> 12) << 11) + (t0 & 2047)
            col0 = ((t0 & 4095) >> 11) * _D
            pltpu.async_copy(
                rows_v.at[buf],
                out_hbm.at[pl.ds(r0, chunk), pl.ds(col0, _D)],
                osem.at[buf],
            )

        def drain_write(buf):
            pltpu.make_async_copy(
                rows_v.at[buf],
                out_hbm.at[pl.ds(0, chunk), pl.ds(0, _D)],
                osem.at[buf],
            ).wait()

        pltpu.sync_copy(idx_hbm.at[pl.ds(w_row0, rows_per_w)], idx_v)
        fire_gathers(0, 0)

        @pl.loop(0, n_chunks, step=_NBUF)
        def trip(g):
            @pl.when(g > 0)
            def _():
                drain_write(1)
            fire_gathers(g + 1, 1)
            drain_gathers(0)
            fire_write(g, 0)

            @pl.when(g < n_chunks - _NBUF)
            def _():
                drain_write(0)
                fire_gathers(g + 2, 0)
            drain_gathers(1)
            fire_write(g + 1, 1)

        drain_write(0)
        drain_write(1)

    return k


def kernel(x, table):
    b, s = x.shape
    h = b // 2
    x_t = jnp.transpose(x)
    t_t = jnp.transpose(table)

    table128 = _tc_transpose_table(t_t)
    table_rm = table128.reshape(-1, _D)

    blk = x_t >> (_PB + 1)
    u = x_t & (2 * _P - 1)
    xg = (blk << (_PB + 1)) + ((u & (_P - 1)) << 1) + (u >> _PB)

    x2d = xg.reshape(-1, _G)
    flat128 = _make_gather(x2d.shape[0])(table_rm, x2d)
    out_t = _tc_untranspose_out(flat128, s, b)
    return jnp.transpose(out_t, (2, 0, 1))


# speedup vs baseline: 2.4481x; 1.1460x over previous
import functools

import jax
import jax.numpy as jnp
from jax import lax
from jax.experimental import pallas as pl
from jax.experimental.pallas import tpu as pltpu
from jax.experimental.pallas import tpu_sc as plsc

_D = 64
_G = 128
_U = 4
_NBUF = 2
_P = 8192
_PB = _P.bit_length() - 1

_info = plsc.get_sparse_core_info()
_NC = _info.num_cores
_NS = _info.num_subcores
_NW = _NC * _NS


def _ttab_body(in_ref, out_ref):
    out_ref[:, :_D] = in_ref[:, :_P].T
    out_ref[:, _D:] = in_ref[:, _P:].T


def _tc_transpose_table(t_t):
    v = t_t.shape[1]
    n_blk = pl.cdiv(v, 2 * _P)
    return pl.pallas_call(
        _ttab_body,
        grid=(n_blk,),
        in_specs=[pl.BlockSpec((_D, 2 * _P), lambda i: (0, i))],
        out_specs=pl.BlockSpec((_P, 2 * _D), lambda i: (i, 0)),
        out_shape=jax.ShapeDtypeStruct((n_blk * _P, 2 * _D), jnp.float32),
        compiler_params=pltpu.CompilerParams(
            dimension_semantics=("parallel",),
        ),
    )(t_t)


_SB = 4


def _tout_body(in_ref, out_ref):
    h = in_ref.shape[0] // _SB
    for i in range(_SB):
        blk = in_ref[pl.ds(i * h, h), :]
        out_ref[i, :, :h] = blk[:, :_D].T
        out_ref[i, :, h:] = blk[:, _D:].T


def _tc_untranspose_out(flat128, s, b):
    h = b // 2
    return pl.pallas_call(
        _tout_body,
        grid=(s // _SB,),
        in_specs=[pl.BlockSpec((_SB * h, 2 * _D), lambda i: (i, 0))],
        out_specs=pl.BlockSpec((_SB, _D, b), lambda i: (i, 0, 0)),
        out_shape=jax.ShapeDtypeStruct((s, _D, b), jnp.float32),
        compiler_params=pltpu.CompilerParams(
            dimension_semantics=("parallel",),
        ),
    )(flat128)


def _make_gather(n_rows: int):
    rows_per_w = n_rows // _NW
    n_chunks = rows_per_w // _U
    assert n_chunks % _NBUF == 0
    mesh = plsc.VectorSubcoreMesh(core_axis_name="c", subcore_axis_name="s")

    chunk = _U * _G

    @functools.partial(
        pl.kernel,
        mesh=mesh,
        out_type=jax.ShapeDtypeStruct((n_rows * _G // 2, 2 * _D), jnp.float32),
        scratch_types=[
            pltpu.VMEM((rows_per_w, _G), jnp.int32),
            pltpu.VMEM((_NBUF, _U * _G, _D), jnp.float32),
            pltpu.SemaphoreType.DMA((_NBUF,)),
            pltpu.SemaphoreType.DMA((_NBUF,)),
        ],
        compiler_params=pltpu.CompilerParams(use_tc_tiling_on_sc=False),
    )
    def k(table_hbm, idx_hbm, out_hbm, idx_v, rows_v, gsem, osem):
        wid = lax.axis_index("s") * _NC + lax.axis_index("c")
        w_row0 = wid * rows_per_w

        def fire_gathers(c, buf):
            for j in range(_U):
                pltpu.async_copy(
                    table_hbm.at[idx_v.at[c * _U + j]],
                    rows_v.at[buf, pl.ds(j * _G, _G)],
                    gsem.at[buf],
                )

        def drain_gathers(buf):
            for j in range(_U):
                pltpu.make_async_copy(
                    table_hbm.at[idx_v.at[0]],
                    rows_v.at[buf, pl.ds(j * _G, _G)],
                    gsem.at[buf],
                ).wait()

        def fire_write(c, buf):
            t0 = (w_row0 + c * _U) * _G
            r0 = ((t0 >> 12) << 11) + (t0 & 2047)
            col0 = ((t0 & 4095) >> 11) * _D
            pltpu.async_copy(
                rows_v.at[buf],
                out_hbm.at[pl.ds(r0, chunk), pl.ds(col0, _D)],
                osem.at[buf],
            )

        def drain_write(buf):
            pltpu.make_async_copy(
                rows_v.at[buf],
                out_hbm.at[pl.ds(0, chunk), pl.ds(0, _D)],
                osem.at[buf],
            ).wait()

        pltpu.sync_copy(idx_hbm.at[pl.ds(w_row0, rows_per_w)], idx_v)
        fire_gathers(0, 0)

        @pl.loop(0, n_chunks, step=_NBUF)
        def trip(g):
            @pl.when(g > 0)
            def _():
                drain_write(1)
            fire_gathers(g + 1, 1)
            drain_gathers(0)
            fire_write(g, 0)

            @pl.when(g < n_chunks - _NBUF)
            def _():
                drain_write(0)
                fire_gathers(g + 2, 0)
            drain_gathers(1)
            fire_write(g + 1, 1)

        drain_write(0)
        drain_write(1)

    return k


def kernel(x, table):
    b, s = x.shape
    h = b // 2
    x_t = jnp.transpose(x)
    t_t = jnp.transpose(table)

    table128 = _tc_transpose_table(t_t)
    table_rm = table128.reshape(-1, _D)

    blk = x_t >> (_PB + 1)
    u = x_t & (2 * _P - 1)
    xg = (blk << (_PB + 1)) + ((u & (_P - 1)) << 1) + (u >> _PB)

    x2d = xg.reshape(-1, _G)
    flat128 = _make_gather(x2d.shape[0])(table_rm, x2d)
    out_t = _tc_untranspose_out(flat128, s, b)
    return jnp.transpose(out_t, (2, 0, 1))


# speedup vs baseline: 2.5033x; 1.0226x over previous
import functools

import jax
import jax.numpy as jnp
from jax import lax
from jax.experimental import pallas as pl
from jax.experimental.pallas import tpu as pltpu
from jax.experimental.pallas import tpu_sc as plsc

_D = 64
_G = 128
_U = 4
_NBUF = 2
_P = 8192
_PB = _P.bit_length() - 1

_info = plsc.get_sparse_core_info()
_NC = _info.num_cores
_NS = _info.num_subcores
_NW = _NC * _NS


def _ttab_body(in_ref, out_ref):
    out_ref[:, :_D] = in_ref[:, :_P].T
    out_ref[:, _D:] = in_ref[:, _P:].T


def _tc_transpose_table(t_t):
    v = t_t.shape[1]
    n_blk = pl.cdiv(v, 2 * _P)
    return pl.pallas_call(
        _ttab_body,
        grid=(n_blk,),
        in_specs=[pl.BlockSpec((_D, 2 * _P), lambda i: (0, i))],
        out_specs=pl.BlockSpec((_P, 2 * _D), lambda i: (i, 0)),
        out_shape=jax.ShapeDtypeStruct((n_blk * _P, 2 * _D), jnp.float32),
        compiler_params=pltpu.CompilerParams(
            dimension_semantics=("parallel",),
        ),
    )(t_t)


_SB = 8


def _tout_body(in_ref, out_ref):
    h = in_ref.shape[0] // _SB
    for i in range(_SB):
        blk = in_ref[pl.ds(i * h, h), :]
        out_ref[i, :, :h] = blk[:, :_D].T
        out_ref[i, :, h:] = blk[:, _D:].T


def _tc_untranspose_out(flat128, s, b):
    h = b // 2
    return pl.pallas_call(
        _tout_body,
        grid=(s // _SB,),
        in_specs=[pl.BlockSpec((_SB * h, 2 * _D), lambda i: (i, 0))],
        out_specs=pl.BlockSpec((_SB, _D, b), lambda i: (i, 0, 0)),
        out_shape=jax.ShapeDtypeStruct((s, _D, b), jnp.float32),
        compiler_params=pltpu.CompilerParams(
            dimension_semantics=("parallel",),
            vmem_limit_bytes=100 * 1024 * 1024,
        ),
    )(flat128)


def _make_gather(n_rows: int):
    rows_per_w = n_rows // _NW
    n_chunks = rows_per_w // _U
    assert n_chunks % _NBUF == 0
    mesh = plsc.VectorSubcoreMesh(core_axis_name="c", subcore_axis_name="s")

    chunk = _U * _G

    @functools.partial(
        pl.kernel,
        mesh=mesh,
        out_type=jax.ShapeDtypeStruct((n_rows * _G // 2, 2 * _D), jnp.float32),
        scratch_types=[
            pltpu.VMEM((rows_per_w, _G), jnp.int32),
            pltpu.VMEM((_NBUF, _U * _G, _D), jnp.float32),
            pltpu.SemaphoreType.DMA((_NBUF,)),
            pltpu.SemaphoreType.DMA((_NBUF,)),
        ],
        compiler_params=pltpu.CompilerParams(use_tc_tiling_on_sc=False),
    )
    def k(table_hbm, idx_hbm, out_hbm, idx_v, rows_v, gsem, osem):
        wid = lax.axis_index("s") * _NC + lax.axis_index("c")
        w_row0 = wid * rows_per_w

        def fire_gathers(c, buf):
            for j in range(_U):
                pltpu.async_copy(
                    table_hbm.at[idx_v.at[c * _U + j]],
                    rows_v.at[buf, pl.ds(j * _G, _G)],
                    gsem.at[buf],
                )

        def drain_gathers(buf):
            for j in range(_U):
                pltpu.make_async_copy(
                    table_hbm.at[idx_v.at[0]],
                    rows_v.at[buf, pl.ds(j * _G, _G)],
                    gsem.at[buf],
                ).wait()

        def fire_write(c, buf):
            t0 = (w_row0 + c * _U) * _G
            r0 = ((t0 >> 12) << 11) + (t0 & 2047)
            col0 = ((t0 & 4095) >> 11) * _D
            pltpu.async_copy(
                rows_v.at[buf],
                out_hbm.at[pl.ds(r0, chunk), pl.ds(col0, _D)],
                osem.at[buf],
            )

        def drain_write(buf):
            pltpu.make_async_copy(
                rows_v.at[buf],
                out_hbm.at[pl.ds(0, chunk), pl.ds(0, _D)],
                osem.at[buf],
            ).wait()

        pltpu.sync_copy(idx_hbm.at[pl.ds(w_row0, rows_per_w)], idx_v)
        fire_gathers(0, 0)

        @pl.loop(0, n_chunks, step=_NBUF)
        def trip(g):
            @pl.when(g > 0)
            def _():
                drain_write(1)
            fire_gathers(g + 1, 1)
            drain_gathers(0)
            fire_write(g, 0)

            @pl.when(g < n_chunks - _NBUF)
            def _():
                drain_write(0)
                fire_gathers(g + 2, 0)
            drain_gathers(1)
            fire_write(g + 1, 1)

        drain_write(0)
        drain_write(1)

    return k


def kernel(x, table):
    b, s = x.shape
    h = b // 2
    x_t = jnp.transpose(x)
    t_t = jnp.transpose(table)

    table128 = _tc_transpose_table(t_t)
    table_rm = table128.reshape(-1, _D)

    blk = x_t >> (_PB + 1)
    u = x_t & (2 * _P - 1)
    xg = (blk << (_PB + 1)) + ((u & (_P - 1)) << 1) + (u >> _PB)

    x2d = xg.reshape(-1, _G)
    flat128 = _make_gather(x2d.shape[0])(table_rm, x2d)
    out_t = _tc_untranspose_out(flat128, s, b)
    return jnp.transpose(out_t, (2, 0, 1))
